# baseline (speedup 1.0000x reference)
import functools

import jax
import jax.numpy as jnp
from jax import lax
from jax.experimental import pallas as pl
from jax.experimental.pallas import tpu as pltpu
from jax.experimental.pallas import tpu_sc as plsc

VOCAB = 100000
EMB = 32
B = 1024
L = 20

NW = 32
B_W = B // NW

TILE_V = 4096
NV = (VOCAB + TILE_V - 1) // TILE_V
VP = NV * TILE_V


def _sc_pool(idx3, table):
    mesh = plsc.VectorSubcoreMesh(core_axis_name="c", subcore_axis_name="s")

    @functools.partial(
        pl.kernel,
        out_type=jax.ShapeDtypeStruct((B, EMB), jnp.float32),
        mesh=mesh,
        scratch_types=[
            pltpu.VMEM((L, B_W), jnp.int32),
            pltpu.VMEM((B_W, EMB), jnp.float32),
            pltpu.SemaphoreType.DMA,
        ],
        compiler_params=pltpu.CompilerParams(use_tc_tiling_on_sc=False),
    )
    def k(idx_hbm, table_hbm, out_hbm, idx_v, acc_v, sem):
        wid = lax.axis_index("s") * 2 + lax.axis_index("c")
        pltpu.sync_copy(idx_hbm.at[wid], idx_v)
        pltpu.async_copy(table_hbm.at[idx_v.at[0]], acc_v, sem).wait()
        descs = [
            pltpu.async_copy(table_hbm.at[idx_v.at[l]], acc_v, sem, add=True)
            for l in range(1, L)
        ]
        for d in descs:
            d.wait()
        pltpu.sync_copy(acc_v, out_hbm.at[pl.ds(wid * B_W, B_W)])

    return k(idx3, table)


def _logits_t(s_ref, w_ref, b_ref):
    acc = lax.dot_general(
        w_ref[...], s_ref[...],
        dimension_numbers=(((1,), (1,)), ((), ())),
        preferred_element_type=jnp.float32,
    )
    return acc + b_ref[...]


def _p1_body(s_ref, w_ref, b_ref, mlse_ref, m_scr, s_scr):
    j = pl.program_id(0)
    logits = (lax.dot_general(
        w_ref[...], s_ref[...],
        dimension_numbers=(((1,), (1,)), ((), ())),
        preferred_element_type=jnp.float32,
    ) + b_ref[...]).astype(jnp.bfloat16)

    @pl.when(j == 0)
    def _():
        m_scr[...] = jnp.full((1, B), -jnp.inf, jnp.float32)
        s_scr[...] = jnp.zeros((1, B), jnp.float32)

    m_old = m_scr[...]
    tile_max = jnp.max(logits, axis=0, keepdims=True).astype(jnp.float32)
    m_new = jnp.maximum(m_old, tile_max)
    s_scr[...] = s_scr[...] * jnp.exp(m_old - m_new) + jnp.sum(
        jnp.exp(logits - m_new.astype(jnp.bfloat16)),
        axis=0, keepdims=True, dtype=jnp.float32)
    m_scr[...] = m_new

    @pl.when(j == NV - 1)
    def _():
        mlse_ref[...] = m_new + jnp.log(s_scr[...])


def _tc_p1(summed, W2, bc):
    return pl.pallas_call(
        _p1_body,
        grid=(NV,),
        in_specs=[
            pl.BlockSpec((B, EMB), lambda j: (0, 0)),
            pl.BlockSpec((TILE_V, EMB), lambda j: (j, 0)),
            pl.BlockSpec((TILE_V, 1), lambda j: (j, 0)),
        ],
        out_specs=pl.BlockSpec((1, B), lambda j: (0, 0)),
        out_shape=jax.ShapeDtypeStruct((1, B), jnp.float32),
        scratch_shapes=[
            pltpu.VMEM((1, B), jnp.float32),
            pltpu.VMEM((1, B), jnp.float32),
        ],
        compiler_params=pltpu.CompilerParams(
            dimension_semantics=("arbitrary",)),
    )(summed, W2, bc)


def _p2_body(s_ref, w_ref, b_ref, mlse_ref, o_ref):
    o_ref[...] = _logits_t(s_ref, w_ref, b_ref) - mlse_ref[...]


def _tc_p2(summed, W2, bc, mlse):
    return pl.pallas_call(
        _p2_body,
        grid=(NV,),
        in_specs=[
            pl.BlockSpec((B, EMB), lambda j: (0, 0)),
            pl.BlockSpec((TILE_V, EMB), lambda j: (j, 0)),
            pl.BlockSpec((TILE_V, 1), lambda j: (j, 0)),
            pl.BlockSpec((1, B), lambda j: (0, 0)),
        ],
        out_specs=pl.BlockSpec((TILE_V, B), lambda j: (j, 0)),
        out_shape=jax.ShapeDtypeStruct((VOCAB, B), jnp.float32),
        compiler_params=pltpu.CompilerParams(
            dimension_semantics=("arbitrary",)),
    )(summed, W2, bc, mlse)


def kernel(inputs, table, W, b):
    idx3 = inputs.reshape(NW, B_W, L).transpose(0, 2, 1)
    summed = _sc_pool(idx3, table)
    W2 = jnp.pad(W, ((0, VP - VOCAB), (0, 0))).astype(jnp.bfloat16)
    bc = jnp.pad(b, (0, VP - VOCAB), constant_values=-1e30).reshape(VP, 1)
    sb = summed.astype(jnp.bfloat16)
    mlse = _tc_p1(sb, W2, bc)
    out_t = _tc_p2(sb, W2, bc, mlse)
    return out_t.T


# speedup vs baseline: 1.0118x; 1.0118x over previous
import functools

import jax
import jax.numpy as jnp
from jax import lax
from jax.experimental import pallas as pl
from jax.experimental.pallas import tpu as pltpu
from jax.experimental.pallas import tpu_sc as plsc

VOCAB = 100000
EMB = 32
B = 1024
L = 20

NW = 32
B_W = B // NW

TILE_V = 4096
NV = (VOCAB + TILE_V - 1) // TILE_V
VP = NV * TILE_V


def _sc_pool(idx3, table):
    mesh = plsc.VectorSubcoreMesh(core_axis_name="c", subcore_axis_name="s")

    @functools.partial(
        pl.kernel,
        out_type=jax.ShapeDtypeStruct((B, EMB), jnp.float32),
        mesh=mesh,
        scratch_types=[
            pltpu.VMEM((L, B_W), jnp.int32),
            pltpu.VMEM((B_W, EMB), jnp.float32),
            pltpu.SemaphoreType.DMA,
        ],
        compiler_params=pltpu.CompilerParams(use_tc_tiling_on_sc=False),
    )
    def k(idx_hbm, table_hbm, out_hbm, idx_v, acc_v, sem):
        wid = lax.axis_index("s") * 2 + lax.axis_index("c")
        pltpu.sync_copy(idx_hbm.at[wid], idx_v)
        pltpu.async_copy(table_hbm.at[idx_v.at[0]], acc_v, sem).wait()
        descs = [
            pltpu.async_copy(table_hbm.at[idx_v.at[l]], acc_v, sem, add=True)
            for l in range(1, L)
        ]
        for d in descs:
            d.wait()
        pltpu.sync_copy(acc_v, out_hbm.at[pl.ds(wid * B_W, B_W)])

    return k(idx3, table)


def _logits_t(s_ref, w_ref, b_ref):
    acc = lax.dot_general(
        w_ref[...], s_ref[...],
        dimension_numbers=(((1,), (1,)), ((), ())),
        preferred_element_type=jnp.float32,
    )
    return acc + b_ref[...]


def _p1_body(s_ref, w_ref, b_ref, mlse_ref, m_scr, s_scr):
    j = pl.program_id(0)
    logits = _logits_t(s_ref, w_ref, b_ref)

    @pl.when(j == 0)
    def _():
        m_scr[...] = jnp.full((1, B), -jnp.inf, jnp.float32)
        s_scr[...] = jnp.zeros((1, B), jnp.float32)

    m_old = m_scr[...]
    m_new = jnp.maximum(m_old, jnp.max(logits, axis=0, keepdims=True))
    s_scr[...] = s_scr[...] * jnp.exp(m_old - m_new) + jnp.sum(
        jnp.exp(logits - m_new), axis=0, keepdims=True)
    m_scr[...] = m_new

    @pl.when(j == NV - 1)
    def _():
        mlse_ref[...] = m_new + jnp.log(s_scr[...])


def _tc_p1(summed, W2, bc):
    return pl.pallas_call(
        _p1_body,
        grid=(NV,),
        in_specs=[
            pl.BlockSpec((B, EMB), lambda j: (0, 0)),
            pl.BlockSpec((TILE_V, EMB), lambda j: (j, 0)),
            pl.BlockSpec((TILE_V, 1), lambda j: (j, 0)),
        ],
        out_specs=pl.BlockSpec((1, B), lambda j: (0, 0)),
        out_shape=jax.ShapeDtypeStruct((1, B), jnp.float32),
        scratch_shapes=[
            pltpu.VMEM((1, B), jnp.float32),
            pltpu.VMEM((1, B), jnp.float32),
        ],
        compiler_params=pltpu.CompilerParams(
            dimension_semantics=("arbitrary",)),
    )(summed, W2, bc)


def _p2_body(s_ref, w_ref, b_ref, mlse_ref, o_ref):
    o_ref[...] = _logits_t(s_ref, w_ref, b_ref) - mlse_ref[...]


def _tc_p2(summed, W2, bc, mlse):
    return pl.pallas_call(
        _p2_body,
        grid=(NV,),
        in_specs=[
            pl.BlockSpec((B, EMB), lambda j: (0, 0)),
            pl.BlockSpec((TILE_V, EMB), lambda j: (j, 0)),
            pl.BlockSpec((TILE_V, 1), lambda j: (j, 0)),
            pl.BlockSpec((1, B), lambda j: (0, 0)),
        ],
        out_specs=pl.BlockSpec((TILE_V, B), lambda j: (j, 0)),
        out_shape=jax.ShapeDtypeStruct((VOCAB, B), jnp.float32),
        compiler_params=pltpu.CompilerParams(
            dimension_semantics=("arbitrary",)),
    )(summed, W2, bc, mlse)


def kernel(inputs, table, W, b):
    idx3 = inputs.reshape(NW, B_W, L).transpose(0, 2, 1)
    summed = _sc_pool(idx3, table)
    W2 = jnp.pad(W, ((0, VP - VOCAB), (0, 0))).astype(jnp.bfloat16)
    bc = jnp.pad(b, (0, VP - VOCAB), constant_values=-1e30).reshape(VP, 1)
    sb = summed.astype(jnp.bfloat16)
    mlse = _tc_p1(sb, W2, bc)
    out_t = _tc_p2(sb, W2, bc, mlse)
    return out_t.T
